# baseline (speedup 1.0000x reference)
import jax
import jax.numpy as jnp
from jax import lax
from jax.experimental import pallas as pl
from jax.experimental.pallas import tpu as pltpu
from jax.experimental.pallas import tpu_sc as plsc

B = 4096
H = 15
W = 15
P = H * W
D = 16
PD = 2380
VOCAB = (PD + 1) ** 2

NC = 2
NS = 16
NW = NC * NS
IMG_PER_W = B // NW
CHUNK = 8
NCHUNK = IMG_PER_W // CHUNK
NPOS = CHUNK * P
NGRP = 113
NPAD = 1808
NIDXROW = 15
NROWS = NIDXROW * 128


def _sc_kernel(sf0_hbm, sf1_hbm, bd0_hbm, bd1_hbm, tab_hbm, out_hbm,
               sf0, sf1, bd0, bd1, idxb, rows, outv, sem):
    wid = lax.axis_index("s") * NC + lax.axis_index("c")
    base_img = wid * IMG_PER_W

    iota = lax.broadcasted_iota(jnp.int32, (16,), 0)
    zeros16 = jnp.zeros((16,), jnp.int32)

    sf0[pl.ds(1792, 16)] = zeros16
    sf1[pl.ds(1792, 16)] = zeros16
    bd0[pl.ds(1792, 16)] = zeros16
    bd1[pl.ds(1792, 16)] = zeros16
    for gg in range(1, 8):
        idxb[NIDXROW - 1, pl.ds(16 * gg, 16)] = zeros16

    @pl.loop(0, NCHUNK)
    def _chunk(c):
        b0 = base_img + c * CHUNK
        e0 = b0 * P

        pltpu.sync_copy(sf0_hbm.at[pl.ds(e0, NPOS)], sf0.at[pl.ds(0, NPOS)])
        pltpu.sync_copy(sf1_hbm.at[pl.ds(e0, NPOS)], sf1.at[pl.ds(0, NPOS)])
        pltpu.sync_copy(bd0_hbm.at[pl.ds(e0, NPOS)], bd0.at[pl.ds(0, NPOS)])
        pltpu.sync_copy(bd1_hbm.at[pl.ds(e0, NPOS)], bd1.at[pl.ds(0, NPOS)])

        for g in range(NGRP):
            f = 16 * g
            s0 = sf0[pl.ds(f, 16)]
            s1 = sf1[pl.ds(f, 16)]
            c0 = bd0[pl.ds(f, 16)]
            c1 = bd1[pl.ds(f, 16)]
            p0 = jnp.where(c0 > 0, PD, s0)
            p1 = jnp.where(c1 > 0, PD, s1)
            iv = p1 * (PD + 1) + p0
            idxb[f // 128, pl.ds(f % 128, 16)] = iv

        @pl.loop(0, NIDXROW)
        def _fire(i):
            pltpu.make_async_copy(
                tab_hbm.at[idxb.at[i]], rows.at[pl.ds(128 * i, 128)], sem
            ).start()

        @pl.loop(0, NIDXROW)
        def _drain(i):
            pltpu.make_async_copy(
                tab_hbm.at[idxb.at[i]], rows.at[pl.ds(128 * i, 128)], sem
            ).wait()

        @pl.loop(0, CHUNK)
        def _transpose(img):
            rowvec = img * 16 + iota
            colvec = jnp.zeros((16,), jnp.int32)
            one = jnp.ones((16,), jnp.int32)
            for p in range(P):
                val = rows[img * P + p]
                plsc.store_scatter(outv, [rowvec, colvec], val)
                colvec = colvec + one

        @pl.loop(0, CHUNK)
        def _out(img):
            pltpu.sync_copy(
                outv.at[pl.ds(16 * img, 16)],
                out_hbm.at[b0 + img],
            )


@jax.jit
def _pcode_embed(sparse_feature_input, board_input, pcode_embedding):
    sf0 = sparse_feature_input[:, 10].reshape(B * P)
    sf1 = sparse_feature_input[:, 11].reshape(B * P)
    bd0 = board_input[:, 0].reshape(B * P)
    bd1 = board_input[:, 1].reshape(B * P)
    mesh = plsc.VectorSubcoreMesh(core_axis_name="c", subcore_axis_name="s")
    run = pl.kernel(
        _sc_kernel,
        out_type=jax.ShapeDtypeStruct((B, D, P), jnp.float32),
        mesh=mesh,
        scratch_types=[
            pltpu.VMEM((NPAD,), jnp.int32),
            pltpu.VMEM((NPAD,), jnp.int32),
            pltpu.VMEM((NPAD,), jnp.int32),
            pltpu.VMEM((NPAD,), jnp.int32),
            pltpu.VMEM((NIDXROW, 128), jnp.int32),
            pltpu.VMEM((NROWS, D), jnp.float32),
            pltpu.VMEM((CHUNK * D, P), jnp.float32),
            pltpu.SemaphoreType.DMA,
        ],
        compiler_params=pltpu.CompilerParams(
            use_tc_tiling_on_sc=False, needs_layout_passes=False
        ),
    )
    out = run(sf0, sf1, bd0, bd1, pcode_embedding)
    return out.reshape(B, D, H, W)


def kernel(sparse_feature_dim, sparse_feature_input, board_input, pcode_embedding):
    del sparse_feature_dim
    return _pcode_embed(sparse_feature_input, board_input, pcode_embedding)
